# baseline (speedup 1.0000x reference)
import jax
import jax.numpy as jnp
from jax import lax
from jax.experimental import pallas as pl
from jax.experimental.pallas import tpu as pltpu
from jax.experimental.pallas import tpu_sc as plsc

N = 10000
NP = 10240
E = 320000
EP = 327680
EROWS = EP // 128
NUM_CAT = 43
EMB = 128
HID = 256
OUT = 64
NUM_GRAPHS = 64
F32 = jnp.float32
I32 = jnp.int32

NC, NS = 2, 16
NW = NC * NS
ROWS_ALL = EROWS // NS
ROWS_HALF = EROWS // NW
NPT = NP // NS
JUNK_NODE = NP - 1


def _qrsqrt(d):
    i = plsc.bitcast(d, I32)
    i = jnp.int32(0x5F3759DF) - (i >> 1)
    y = plsc.bitcast(i, F32)
    for _ in range(3):
        y = y * (1.5 - 0.5 * d * y * y)
    return y


def _sc_a(src2d, dst2d, x):
    mesh = plsc.VectorSubcoreMesh(core_axis_name="c", subcore_axis_name="s")

    def body(src_hbm, dst_hbm, x_hbm, outB_hbm, dinv_hbm,
             b_sh, deg_sh, dinv_sh,
             x_v, dinv_v, dstv_all, srcv, dstv, idxv, valv,
             selfidx, selfval, ones_row, zeros_v, sem):
        c = lax.axis_index("c")
        s = lax.axis_index("s")
        wid = s * NC + c
        node0 = s * NPT

        for j in range(8):
            ones_row[0, pl.ds(j * 16, 16)] = jnp.ones((16,), F32)

        def zero_step(r, carry):
            zeros_v[pl.ds(r * 16, 16)] = jnp.zeros((16,), F32)
            return carry
        lax.fori_loop(0, 160, zero_step, None)

        pltpu.sync_copy(x_hbm, x_v.at[pl.ds(0, N)])
        for j in range(15):
            x_v[pl.ds(N + j * 16, 16)] = jnp.zeros((16,), I32)

        pltpu.sync_copy(zeros_v.at[pl.ds(0, NPT)],
                        deg_sh.at[pl.ds(node0, NPT)])
        for k in range(16):
            pltpu.sync_copy(zeros_v, b_sh.at[pl.ds(node0 * 64 + k * 2560,
                                                   2560)])
        plsc.subcore_barrier()

        pltpu.sync_copy(dst_hbm.at[pl.ds(s * ROWS_ALL, ROWS_ALL)], dstv_all)

        def deg_step(r, carry):
            pltpu.async_copy(ones_row.at[0], deg_sh.at[dstv_all.at[r]], sem,
                             add=True)
            return carry
        lax.fori_loop(0, ROWS_ALL, deg_step, None)

        def deg_drain(r, carry):
            pltpu.make_async_copy(ones_row.at[0], deg_sh.at[dstv_all.at[r]],
                                  sem).wait()
            return carry
        lax.fori_loop(0, ROWS_ALL, deg_drain, None)
        plsc.subcore_barrier()

        pltpu.sync_copy(deg_sh.at[pl.ds(node0, NPT)],
                        dinv_v.at[pl.ds(node0, NPT)])
        for k in range(NPT // 16):
            d = dinv_v[pl.ds(node0 + k * 16, 16)] + 1.0
            dinv_v[pl.ds(node0 + k * 16, 16)] = _qrsqrt(d)
        pltpu.sync_copy(dinv_v.at[pl.ds(node0, NPT)],
                        dinv_sh.at[pl.ds(node0, NPT)])

        @pl.when(c == 0)
        def _():
            pltpu.sync_copy(dinv_v.at[pl.ds(node0, NPT)],
                            dinv_hbm.at[pl.ds(node0, NPT)])
        plsc.subcore_barrier()
        pltpu.sync_copy(dinv_sh, dinv_v)

        er0 = wid * ROWS_HALF
        pltpu.sync_copy(src_hbm.at[pl.ds(er0, ROWS_HALF)], srcv)
        pltpu.sync_copy(dst_hbm.at[pl.ds(er0, ROWS_HALF)], dstv)

        def b_step(r, carry):
            for j in range(8):
                s16 = srcv[r, pl.ds(j * 16, 16)]
                d16 = dstv[r, pl.ds(j * 16, 16)]
                xs = plsc.load_gather(x_v, [s16])
                dv = plsc.load_gather(dinv_v, [s16])
                idxv[r, pl.ds(j * 16, 16)] = d16 * 64 + xs
                valv[r, pl.ds(j * 16, 16)] = dv
            pltpu.async_copy(valv.at[r], b_sh.at[idxv.at[r]], sem, add=True)
            return carry
        lax.fori_loop(0, ROWS_HALF, b_step, None)

        @pl.when(c == 0)
        def _():
            def self_step(r, carry):
                for j in range(8):
                    nb = node0 + r * 128 + j * 16
                    n16 = lax.iota(I32, 16) + nb
                    xs = x_v[pl.ds(nb, 16)]
                    selfidx[r, pl.ds(j * 16, 16)] = n16 * 64 + xs
                    selfval[r, pl.ds(j * 16, 16)] = dinv_v[pl.ds(nb, 16)]
                pltpu.async_copy(selfval.at[r], b_sh.at[selfidx.at[r]], sem,
                                 add=True)
                return carry
            lax.fori_loop(0, 5, self_step, None)

        def b_drain(r, carry):
            pltpu.make_async_copy(valv.at[r], b_sh.at[idxv.at[r]], sem).wait()
            return carry
        lax.fori_loop(0, ROWS_HALF, b_drain, None)

        @pl.when(c == 0)
        def _():
            def self_drain(r, carry):
                pltpu.make_async_copy(selfval.at[r], b_sh.at[selfidx.at[r]],
                                      sem).wait()
                return carry
            lax.fori_loop(0, 5, self_drain, None)
        plsc.subcore_barrier()

        pltpu.sync_copy(b_sh.at[pl.ds(node0 * 64, NPT * 64)],
                        outB_hbm.at[c, pl.ds(node0 * 64, NPT * 64)])

    f = pl.kernel(
        body,
        out_type=[jax.ShapeDtypeStruct((NC, NP * 64), F32),
                  jax.ShapeDtypeStruct((NP,), F32)],
        mesh=mesh,
        compiler_params=pltpu.CompilerParams(needs_layout_passes=False,
                                             use_tc_tiling_on_sc=False),
        scratch_types=[
            pltpu.VMEM_SHARED((NP * 64,), F32),
            pltpu.VMEM_SHARED((NP,), F32),
            pltpu.VMEM_SHARED((NP,), F32),
            pltpu.VMEM((NP,), I32),
            pltpu.VMEM((NP,), F32),
            pltpu.VMEM((ROWS_ALL, 128), I32),
            pltpu.VMEM((ROWS_HALF, 128), I32),
            pltpu.VMEM((ROWS_HALF, 128), I32),
            pltpu.VMEM((ROWS_HALF, 128), I32),
            pltpu.VMEM((ROWS_HALF, 128), F32),
            pltpu.VMEM((5, 128), I32),
            pltpu.VMEM((5, 128), F32),
            pltpu.VMEM((1, 128), F32),
            pltpu.VMEM((2560,), F32),
            pltpu.SemaphoreType.DMA,
        ],
    )
    return f(src2d, dst2d, x)


FW = 64


def _sc_b(src2d, dst2d, g):
    mesh = plsc.VectorSubcoreMesh(core_axis_name="c", subcore_axis_name="s")

    def body(src_hbm, dst_hbm, g_hbm, outS_hbm,
             s_sh, srcv, dstv, rows0, rows1, rows2, rows3, zrows,
             sem0, sem1, sem2, sem3):
        c = lax.axis_index("c")
        s = lax.axis_index("s")
        wid = s * NC + c
        node0 = s * NPT
        bufs = (rows0, rows1, rows2, rows3)
        sems = (sem0, sem1, sem2, sem3)

        def zrow_step(r, carry):
            for j in range(FW // 16):
                zrows[r, pl.ds(j * 16, 16)] = jnp.zeros((16,), F32)
            return carry
        lax.fori_loop(0, 8, zrow_step, None)

        def zinit_step(k, carry):
            pltpu.sync_copy(zrows, s_sh.at[pl.ds(node0 + k * 8, 8)])
            return carry
        lax.fori_loop(0, NPT // 8, zinit_step, None)
        plsc.subcore_barrier()

        er0 = wid * ROWS_HALF
        pltpu.sync_copy(src_hbm.at[pl.ds(er0, ROWS_HALF)], srcv)
        pltpu.sync_copy(dst_hbm.at[pl.ds(er0, ROWS_HALF)], dstv)

        for jj in range(3):
            pltpu.async_copy(g_hbm.at[srcv.at[jj]], bufs[jj], sems[jj])

        def step(i, carry):
            r0 = i * 4
            for jj in range(4):
                r = r0 + jj
                pf = r + 3
                @pl.when(pf < ROWS_HALF)
                def _():
                    pltpu.async_copy(g_hbm.at[srcv.at[pf]], bufs[(jj + 3) % 4],
                                     sems[(jj + 3) % 4])
                pltpu.make_async_copy(g_hbm.at[srcv.at[r]], bufs[jj],
                                      sems[jj]).wait()
                pltpu.sync_copy(bufs[jj], s_sh.at[dstv.at[r]], add=True)
            return carry
        lax.fori_loop(0, ROWS_HALF // 4, step, None)
        plsc.subcore_barrier()

        for k in range(5):
            sl = pl.ds(node0 + k * 128, 128)
            pltpu.sync_copy(s_sh.at[sl], rows0)
            pltpu.sync_copy(rows0, outS_hbm.at[c, sl])

    f = pl.kernel(
        body,
        out_type=jax.ShapeDtypeStruct((NC, NP, FW), F32),
        mesh=mesh,
        compiler_params=pltpu.CompilerParams(needs_layout_passes=False,
                                             use_tc_tiling_on_sc=False),
        scratch_types=[
            pltpu.VMEM_SHARED((NP, FW), F32),
            pltpu.VMEM((ROWS_HALF, 128), I32),
            pltpu.VMEM((ROWS_HALF, 128), I32),
            pltpu.VMEM((128, FW), F32),
            pltpu.VMEM((128, FW), F32),
            pltpu.VMEM((128, FW), F32),
            pltpu.VMEM((128, FW), F32),
            pltpu.VMEM((8, FW), F32),
            pltpu.SemaphoreType.DMA,
            pltpu.SemaphoreType.DMA,
            pltpu.SemaphoreType.DMA,
            pltpu.SemaphoreType.DMA,
        ],
    )
    return f(src2d, dst2d, g)


def _tc1_body(b0_ref, b1_ref, dinv_ref, emb_ref, w1_ref, bias1_ref, w2_ref,
              g_ref, h1p_ref):
    i = pl.program_id(0)

    @pl.when(i == 0)
    def _():
        h1p_ref[...] = jnp.dot(emb_ref[...], w1_ref[...],
                               preferred_element_type=F32)
    B = b0_ref[...] + b1_ref[...]
    t = jnp.dot(B, h1p_ref[...], preferred_element_type=F32)
    t = jax.nn.relu(dinv_ref[...] * t + bias1_ref[...])
    g_ref[...] = jnp.dot(t, w2_ref[...],
                         preferred_element_type=F32) * dinv_ref[...]


def _tc1(B0, B1, dinv2d, emb_pad, W1, b1, W2):
    BLK = 1024
    return pl.pallas_call(
        _tc1_body,
        grid=(NP // BLK,),
        in_specs=[
            pl.BlockSpec((BLK, 64), lambda i: (i, 0)),
            pl.BlockSpec((BLK, 64), lambda i: (i, 0)),
            pl.BlockSpec((BLK, 1), lambda i: (i, 0)),
            pl.BlockSpec((64, EMB), lambda i: (0, 0)),
            pl.BlockSpec((EMB, HID), lambda i: (0, 0)),
            pl.BlockSpec((1, HID), lambda i: (0, 0)),
            pl.BlockSpec((HID, OUT), lambda i: (0, 0)),
        ],
        out_specs=pl.BlockSpec((BLK, OUT), lambda i: (i, 0)),
        out_shape=jax.ShapeDtypeStruct((NP, OUT), F32),
        scratch_shapes=[pltpu.VMEM((64, HID), F32)],
    )(B0, B1, dinv2d, emb_pad, W1, b1, W2)


def _tc2_body(s0_ref, s1_ref, g_ref, dinv_ref, b2_ref, wout_ref, bout_ref,
              batch_ref, out_ref, macc, cacc):
    i = pl.program_id(0)
    ng = pl.num_programs(0)

    @pl.when(i == 0)
    def _():
        macc[...] = jnp.zeros_like(macc)
        cacc[...] = jnp.zeros_like(cacc)
    sm = s0_ref[...] + s1_ref[...] + g_ref[...]
    a2 = dinv_ref[...] * sm + b2_ref[...]
    h3 = jax.nn.relu(jnp.dot(a2, wout_ref[...],
                             preferred_element_type=F32) + bout_ref[...])
    M = (batch_ref[...] == lax.broadcasted_iota(I32, (1, NUM_GRAPHS), 1)
         ).astype(F32)
    macc[...] += lax.dot_general(M, h3, (((0,), (0,)), ((), ())),
                                 preferred_element_type=F32)
    cacc[...] += lax.dot_general(M, jnp.ones_like(h3), (((0,), (0,)), ((), ())),
                                 preferred_element_type=F32)

    @pl.when(i == ng - 1)
    def _():
        out_ref[...] = macc[...] / jnp.maximum(cacc[...], 1.0)


def _tc2(S0, S1, G, dinv2d, b2, Wout, bout, batch2d):
    BLK = 1024
    return pl.pallas_call(
        _tc2_body,
        grid=(NP // BLK,),
        in_specs=[
            pl.BlockSpec((BLK, OUT), lambda i: (i, 0)),
            pl.BlockSpec((BLK, OUT), lambda i: (i, 0)),
            pl.BlockSpec((BLK, OUT), lambda i: (i, 0)),
            pl.BlockSpec((BLK, 1), lambda i: (i, 0)),
            pl.BlockSpec((1, OUT), lambda i: (0, 0)),
            pl.BlockSpec((OUT, OUT), lambda i: (0, 0)),
            pl.BlockSpec((1, OUT), lambda i: (0, 0)),
            pl.BlockSpec((BLK, 1), lambda i: (i, 0)),
        ],
        out_specs=pl.BlockSpec((NUM_GRAPHS, NUM_GRAPHS), lambda i: (0, 0)),
        out_shape=jax.ShapeDtypeStruct((NUM_GRAPHS, NUM_GRAPHS), F32),
        scratch_shapes=[pltpu.VMEM((NUM_GRAPHS, NUM_GRAPHS), F32),
                        pltpu.VMEM((NUM_GRAPHS, NUM_GRAPHS), F32)],
    )(S0, S1, G, dinv2d, b2, Wout, bout, batch2d)


def kernel(x, edge_index, edge_attr, batch, emb_table, W1, b1, W2, b2,
           Wout, bout):
    del edge_attr
    x = x.astype(I32)
    src = edge_index[0].astype(I32)
    dst = edge_index[1].astype(I32)
    pad = EP - E
    pidx = jnp.arange(pad, dtype=I32)
    src2d = jnp.concatenate([src, pidx % N]).reshape(EROWS, 128)
    dst2d = jnp.concatenate([dst, N + pidx % (NP - N)]).reshape(EROWS, 128)

    outB, dinv = _sc_a(src2d, dst2d, x)
    B0 = outB[0].reshape(NP, 64)
    B1 = outB[1].reshape(NP, 64)
    dinv2d = dinv.reshape(NP, 1)

    emb_pad = jnp.zeros((64, EMB), F32).at[:NUM_CAT].set(emb_table)
    g = _tc1(B0, B1, dinv2d, emb_pad, W1, b1.reshape(1, HID), W2)

    outS = _sc_b(src2d, dst2d, g)

    batch_pad = jnp.concatenate([batch.astype(I32),
                                 jnp.full((NP - N,), NUM_GRAPHS, I32)])
    out = _tc2(outS[0], outS[1], g, dinv2d, b2.reshape(1, OUT), Wout,
               bout.reshape(1, OUT), batch_pad.reshape(NP, 1))
    return out


# speedup vs baseline: 1.2470x; 1.2470x over previous
import jax
import jax.numpy as jnp
from jax import lax
from jax.experimental import pallas as pl
from jax.experimental.pallas import tpu as pltpu
from jax.experimental.pallas import tpu_sc as plsc

N = 10000
NP = 10240
E = 320000
EP = 327680
EROWS = EP // 128
NUM_CAT = 43
EMB = 128
HID = 256
OUT = 64
NUM_GRAPHS = 64
F32 = jnp.float32
I32 = jnp.int32

NC, NS = 2, 16
NW = NC * NS
ROWS_ALL = EROWS // NS
ROWS_HALF = EROWS // NW
NPT = NP // NS
JUNK_NODE = NP - 1


def _qrsqrt(d):
    i = plsc.bitcast(d, I32)
    i = jnp.int32(0x5F3759DF) - (i >> 1)
    y = plsc.bitcast(i, F32)
    for _ in range(3):
        y = y * (1.5 - 0.5 * d * y * y)
    return y


def _sc_a(src2d, dst2d, x):
    mesh = plsc.VectorSubcoreMesh(core_axis_name="c", subcore_axis_name="s")

    def body(src_hbm, dst_hbm, x_hbm, outB_hbm, dinv_hbm,
             b_sh, deg_sh, dinv_sh,
             x_v, dinv_v, dstv_all, srcv, dstv, idxv, valv,
             selfidx, selfval, ones_row, zeros_v, sem):
        c = lax.axis_index("c")
        s = lax.axis_index("s")
        wid = s * NC + c
        node0 = s * NPT

        for j in range(8):
            ones_row[0, pl.ds(j * 16, 16)] = jnp.ones((16,), F32)

        def zero_step(r, carry):
            zeros_v[pl.ds(r * 16, 16)] = jnp.zeros((16,), F32)
            return carry
        lax.fori_loop(0, 160, zero_step, None)

        pltpu.sync_copy(x_hbm, x_v.at[pl.ds(0, N)])
        for j in range(15):
            x_v[pl.ds(N + j * 16, 16)] = jnp.zeros((16,), I32)

        pltpu.sync_copy(zeros_v.at[pl.ds(0, NPT)],
                        deg_sh.at[pl.ds(node0, NPT)])
        for k in range(16):
            pltpu.sync_copy(zeros_v, b_sh.at[pl.ds(node0 * 64 + k * 2560,
                                                   2560)])
        plsc.subcore_barrier()

        pltpu.sync_copy(dst_hbm.at[pl.ds(s * ROWS_ALL, ROWS_ALL)], dstv_all)

        def deg_step(r, carry):
            pltpu.async_copy(ones_row.at[0], deg_sh.at[dstv_all.at[r]], sem,
                             add=True)
            return carry
        lax.fori_loop(0, ROWS_ALL, deg_step, None)

        def deg_drain(r, carry):
            pltpu.make_async_copy(ones_row.at[0], deg_sh.at[dstv_all.at[r]],
                                  sem).wait()
            return carry
        lax.fori_loop(0, ROWS_ALL, deg_drain, None)
        plsc.subcore_barrier()

        pltpu.sync_copy(deg_sh.at[pl.ds(node0, NPT)],
                        dinv_v.at[pl.ds(node0, NPT)])
        for k in range(NPT // 16):
            d = dinv_v[pl.ds(node0 + k * 16, 16)] + 1.0
            dinv_v[pl.ds(node0 + k * 16, 16)] = _qrsqrt(d)
        pltpu.sync_copy(dinv_v.at[pl.ds(node0, NPT)],
                        dinv_sh.at[pl.ds(node0, NPT)])

        @pl.when(c == 0)
        def _():
            pltpu.sync_copy(dinv_v.at[pl.ds(node0, NPT)],
                            dinv_hbm.at[pl.ds(node0, NPT)])
        plsc.subcore_barrier()
        pltpu.sync_copy(dinv_sh, dinv_v)

        er0 = wid * ROWS_HALF
        pltpu.sync_copy(src_hbm.at[pl.ds(er0, ROWS_HALF)], srcv)
        pltpu.sync_copy(dst_hbm.at[pl.ds(er0, ROWS_HALF)], dstv)

        def b_step(r, carry):
            for j in range(8):
                s16 = srcv[r, pl.ds(j * 16, 16)]
                d16 = dstv[r, pl.ds(j * 16, 16)]
                xs = plsc.load_gather(x_v, [s16])
                dv = plsc.load_gather(dinv_v, [s16])
                idxv[r, pl.ds(j * 16, 16)] = xs * NP + d16
                valv[r, pl.ds(j * 16, 16)] = dv
            pltpu.async_copy(valv.at[r], b_sh.at[idxv.at[r]], sem, add=True)
            return carry
        lax.fori_loop(0, ROWS_HALF, b_step, None)

        @pl.when(c == 0)
        def _():
            def self_step(r, carry):
                for j in range(8):
                    nb = node0 + r * 128 + j * 16
                    n16 = lax.iota(I32, 16) + nb
                    xs = x_v[pl.ds(nb, 16)]
                    selfidx[r, pl.ds(j * 16, 16)] = xs * NP + n16
                    selfval[r, pl.ds(j * 16, 16)] = dinv_v[pl.ds(nb, 16)]
                pltpu.async_copy(selfval.at[r], b_sh.at[selfidx.at[r]], sem,
                                 add=True)
                return carry
            lax.fori_loop(0, 5, self_step, None)

        def b_drain(r, carry):
            pltpu.make_async_copy(valv.at[r], b_sh.at[idxv.at[r]], sem).wait()
            return carry
        lax.fori_loop(0, ROWS_HALF, b_drain, None)

        @pl.when(c == 0)
        def _():
            def self_drain(r, carry):
                pltpu.make_async_copy(selfval.at[r], b_sh.at[selfidx.at[r]],
                                      sem).wait()
                return carry
            lax.fori_loop(0, 5, self_drain, None)
        plsc.subcore_barrier()

        for cat in range(64):
            pltpu.async_copy(b_sh.at[pl.ds(cat * NP + node0, NPT)],
                             outB_hbm.at[c, cat, pl.ds(node0, NPT)], sem)
        for cat in range(64):
            pltpu.make_async_copy(b_sh.at[pl.ds(cat * NP + node0, NPT)],
                                  outB_hbm.at[c, cat, pl.ds(node0, NPT)],
                                  sem).wait()

    f = pl.kernel(
        body,
        out_type=[jax.ShapeDtypeStruct((NC, 64, NP), F32),
                  jax.ShapeDtypeStruct((NP,), F32)],
        mesh=mesh,
        compiler_params=pltpu.CompilerParams(needs_layout_passes=False),
        scratch_types=[
            pltpu.VMEM_SHARED((NP * 64,), F32),
            pltpu.VMEM_SHARED((NP,), F32),
            pltpu.VMEM_SHARED((NP,), F32),
            pltpu.VMEM((NP,), I32),
            pltpu.VMEM((NP,), F32),
            pltpu.VMEM((ROWS_ALL, 128), I32),
            pltpu.VMEM((ROWS_HALF, 128), I32),
            pltpu.VMEM((ROWS_HALF, 128), I32),
            pltpu.VMEM((ROWS_HALF, 128), I32),
            pltpu.VMEM((ROWS_HALF, 128), F32),
            pltpu.VMEM((5, 128), I32),
            pltpu.VMEM((5, 128), F32),
            pltpu.VMEM((1, 128), F32),
            pltpu.VMEM((2560,), F32),
            pltpu.SemaphoreType.DMA,
        ],
    )
    return f(src2d, dst2d, x)


FW = 64


def _sc_b(src2d, dst2d, g):
    mesh = plsc.VectorSubcoreMesh(core_axis_name="c", subcore_axis_name="s")

    def body(src_hbm, dst_hbm, g_hbm, outS_hbm,
             s_sh, srcv, dstv, rows0, rows1, rows2, rows3, zrows,
             sem0, sem1, sem2, sem3):
        c = lax.axis_index("c")
        s = lax.axis_index("s")
        wid = s * NC + c
        node0 = s * NPT
        bufs = (rows0, rows1, rows2, rows3)
        sems = (sem0, sem1, sem2, sem3)

        def zrow_step(r, carry):
            for j in range(FW // 16):
                zrows[r, pl.ds(j * 16, 16)] = jnp.zeros((16,), F32)
            return carry
        lax.fori_loop(0, 8, zrow_step, None)

        def zinit_step(k, carry):
            pltpu.sync_copy(zrows, s_sh.at[pl.ds(node0 + k * 8, 8)])
            return carry
        lax.fori_loop(0, NPT // 8, zinit_step, None)
        plsc.subcore_barrier()

        er0 = wid * ROWS_HALF
        pltpu.sync_copy(src_hbm.at[pl.ds(er0, ROWS_HALF)], srcv)
        pltpu.sync_copy(dst_hbm.at[pl.ds(er0, ROWS_HALF)], dstv)

        for jj in range(3):
            pltpu.async_copy(g_hbm.at[srcv.at[jj]], bufs[jj], sems[jj])

        def step(i, carry):
            r0 = i * 4
            for jj in range(4):
                r = r0 + jj
                pf = r + 3
                @pl.when(pf < ROWS_HALF)
                def _():
                    pltpu.async_copy(g_hbm.at[srcv.at[pf]], bufs[(jj + 3) % 4],
                                     sems[(jj + 3) % 4])
                pltpu.make_async_copy(g_hbm.at[srcv.at[r]], bufs[jj],
                                      sems[jj]).wait()
                pltpu.sync_copy(bufs[jj], s_sh.at[dstv.at[r]], add=True)
            return carry
        lax.fori_loop(0, ROWS_HALF // 4, step, None)
        plsc.subcore_barrier()

        for k in range(5):
            sl = pl.ds(node0 + k * 128, 128)
            pltpu.sync_copy(s_sh.at[sl], rows0)
            pltpu.sync_copy(rows0, outS_hbm.at[c, sl])

    f = pl.kernel(
        body,
        out_type=jax.ShapeDtypeStruct((NC, NP, FW), F32),
        mesh=mesh,
        compiler_params=pltpu.CompilerParams(needs_layout_passes=False,
                                             use_tc_tiling_on_sc=False),
        scratch_types=[
            pltpu.VMEM_SHARED((NP, FW), F32),
            pltpu.VMEM((ROWS_HALF, 128), I32),
            pltpu.VMEM((ROWS_HALF, 128), I32),
            pltpu.VMEM((128, FW), F32),
            pltpu.VMEM((128, FW), F32),
            pltpu.VMEM((128, FW), F32),
            pltpu.VMEM((128, FW), F32),
            pltpu.VMEM((8, FW), F32),
            pltpu.SemaphoreType.DMA,
            pltpu.SemaphoreType.DMA,
            pltpu.SemaphoreType.DMA,
            pltpu.SemaphoreType.DMA,
        ],
    )
    return f(src2d, dst2d, g)


def _tc1_body(b0_ref, b1_ref, dinv_ref, emb_ref, w1_ref, bias1_ref, w2_ref,
              g_ref, h1p_ref):
    i = pl.program_id(0)

    @pl.when(i == 0)
    def _():
        h1p_ref[...] = jnp.dot(emb_ref[...], w1_ref[...],
                               preferred_element_type=F32)
    Bt = b0_ref[0] + b1_ref[0]
    t = lax.dot_general(Bt, h1p_ref[...], (((0,), (0,)), ((), ())),
                        preferred_element_type=F32)
    t = jax.nn.relu(dinv_ref[...] * t + bias1_ref[...])
    g_ref[...] = jnp.dot(t, w2_ref[...],
                         preferred_element_type=F32) * dinv_ref[...]


def _tc1(outB, dinv2d, emb_pad, W1, b1, W2):
    BLK = 1024
    return pl.pallas_call(
        _tc1_body,
        grid=(NP // BLK,),
        in_specs=[
            pl.BlockSpec((1, 64, BLK), lambda i: (0, 0, i)),
            pl.BlockSpec((1, 64, BLK), lambda i: (1, 0, i)),
            pl.BlockSpec((BLK, 1), lambda i: (i, 0)),
            pl.BlockSpec((64, EMB), lambda i: (0, 0)),
            pl.BlockSpec((EMB, HID), lambda i: (0, 0)),
            pl.BlockSpec((1, HID), lambda i: (0, 0)),
            pl.BlockSpec((HID, OUT), lambda i: (0, 0)),
        ],
        out_specs=pl.BlockSpec((BLK, OUT), lambda i: (i, 0)),
        out_shape=jax.ShapeDtypeStruct((NP, OUT), F32),
        scratch_shapes=[pltpu.VMEM((64, HID), F32)],
    )(outB, outB, dinv2d, emb_pad, W1, b1, W2)


def _tc2_body(s0_ref, s1_ref, g_ref, dinv_ref, b2_ref, wout_ref, bout_ref,
              batch_ref, out_ref, macc, cacc):
    i = pl.program_id(0)
    ng = pl.num_programs(0)

    @pl.when(i == 0)
    def _():
        macc[...] = jnp.zeros_like(macc)
        cacc[...] = jnp.zeros_like(cacc)
    sm = s0_ref[...] + s1_ref[...] + g_ref[...]
    a2 = dinv_ref[...] * sm + b2_ref[...]
    h3 = jax.nn.relu(jnp.dot(a2, wout_ref[...],
                             preferred_element_type=F32) + bout_ref[...])
    M = (batch_ref[...] == lax.broadcasted_iota(I32, (1, NUM_GRAPHS), 1)
         ).astype(F32)
    macc[...] += lax.dot_general(M, h3, (((0,), (0,)), ((), ())),
                                 preferred_element_type=F32)
    cacc[...] += lax.dot_general(M, jnp.ones_like(h3), (((0,), (0,)), ((), ())),
                                 preferred_element_type=F32)

    @pl.when(i == ng - 1)
    def _():
        out_ref[...] = macc[...] / jnp.maximum(cacc[...], 1.0)


def _tc2(S0, S1, G, dinv2d, b2, Wout, bout, batch2d):
    BLK = 1024
    return pl.pallas_call(
        _tc2_body,
        grid=(NP // BLK,),
        in_specs=[
            pl.BlockSpec((BLK, OUT), lambda i: (i, 0)),
            pl.BlockSpec((BLK, OUT), lambda i: (i, 0)),
            pl.BlockSpec((BLK, OUT), lambda i: (i, 0)),
            pl.BlockSpec((BLK, 1), lambda i: (i, 0)),
            pl.BlockSpec((1, OUT), lambda i: (0, 0)),
            pl.BlockSpec((OUT, OUT), lambda i: (0, 0)),
            pl.BlockSpec((1, OUT), lambda i: (0, 0)),
            pl.BlockSpec((BLK, 1), lambda i: (i, 0)),
        ],
        out_specs=pl.BlockSpec((NUM_GRAPHS, NUM_GRAPHS), lambda i: (0, 0)),
        out_shape=jax.ShapeDtypeStruct((NUM_GRAPHS, NUM_GRAPHS), F32),
        scratch_shapes=[pltpu.VMEM((NUM_GRAPHS, NUM_GRAPHS), F32),
                        pltpu.VMEM((NUM_GRAPHS, NUM_GRAPHS), F32)],
    )(S0, S1, G, dinv2d, b2, Wout, bout, batch2d)


def kernel(x, edge_index, edge_attr, batch, emb_table, W1, b1, W2, b2,
           Wout, bout):
    del edge_attr
    x = x.astype(I32)
    src = edge_index[0].astype(I32)
    dst = edge_index[1].astype(I32)
    pad = EP - E
    pidx = jnp.arange(pad, dtype=I32)
    src2d = jnp.concatenate([src, pidx % N]).reshape(EROWS, 128)
    dst2d = jnp.concatenate([dst, N + pidx % (NP - N)]).reshape(EROWS, 128)

    outB, dinv = _sc_a(src2d, dst2d, x)
    dinv2d = dinv.reshape(NP, 1)

    emb_pad = jnp.zeros((64, EMB), F32).at[:NUM_CAT].set(emb_table)
    g = _tc1(outB, dinv2d, emb_pad, W1, b1.reshape(1, HID), W2)

    outS = _sc_b(src2d, dst2d, g)

    batch_pad = jnp.concatenate([batch.astype(I32),
                                 jnp.full((NP - N,), NUM_GRAPHS, I32)])
    out = _tc2(outS[0], outS[1], g, dinv2d, b2.reshape(1, OUT), Wout,
               bout.reshape(1, OUT), batch_pad.reshape(NP, 1))
    return out


# speedup vs baseline: 1.3390x; 1.0738x over previous
import jax
import jax.numpy as jnp
from jax import lax
from jax.experimental import pallas as pl
from jax.experimental.pallas import tpu as pltpu
from jax.experimental.pallas import tpu_sc as plsc

N = 10000
NP = 10240
E = 320000
EP = 327680
EROWS = EP // 128
NUM_CAT = 43
EMB = 128
HID = 256
OUT = 64
NUM_GRAPHS = 64
F32 = jnp.float32
I32 = jnp.int32

NC, NS = 2, 16
NW = NC * NS
ROWS_ALL = EROWS // NS
ROWS_HALF = EROWS // NW
NPT = NP // NS
JUNK_NODE = NP - 1


def _qrsqrt(d):
    i = plsc.bitcast(d, I32)
    i = jnp.int32(0x5F3759DF) - (i >> 1)
    y = plsc.bitcast(i, F32)
    for _ in range(3):
        y = y * (1.5 - 0.5 * d * y * y)
    return y


def _sc_a(src2d, dst2d, x):
    mesh = plsc.VectorSubcoreMesh(core_axis_name="c", subcore_axis_name="s")

    def body(src_hbm, dst_hbm, x_hbm, outB_hbm, dinv_hbm,
             b_sh, deg_sh, dinv_sh,
             x_v, dinv_v, dstv_all, srcv, dstv, idxv, valv,
             selfidx, selfval, ones_row, zeros_v, sem):
        c = lax.axis_index("c")
        s = lax.axis_index("s")
        wid = s * NC + c
        node0 = s * NPT

        for j in range(8):
            ones_row[0, pl.ds(j * 16, 16)] = jnp.ones((16,), F32)

        def zero_step(r, carry):
            zeros_v[pl.ds(r * 16, 16)] = jnp.zeros((16,), F32)
            return carry
        lax.fori_loop(0, 160, zero_step, None)

        pltpu.sync_copy(x_hbm, x_v.at[pl.ds(0, N)])
        for j in range(15):
            x_v[pl.ds(N + j * 16, 16)] = jnp.zeros((16,), I32)

        pltpu.sync_copy(zeros_v.at[pl.ds(0, NPT)],
                        deg_sh.at[pl.ds(node0, NPT)])
        for k in range(16):
            pltpu.sync_copy(zeros_v, b_sh.at[pl.ds(node0 * 64 + k * 2560,
                                                   2560)])
        plsc.subcore_barrier()

        pltpu.sync_copy(dst_hbm.at[pl.ds(s * ROWS_ALL, ROWS_ALL)], dstv_all)

        def deg_step(r, carry):
            pltpu.async_copy(ones_row.at[0], deg_sh.at[dstv_all.at[r]], sem,
                             add=True)
            return carry
        lax.fori_loop(0, ROWS_ALL, deg_step, None)

        def deg_drain(r, carry):
            pltpu.make_async_copy(ones_row.at[0], deg_sh.at[dstv_all.at[r]],
                                  sem).wait()
            return carry
        lax.fori_loop(0, ROWS_ALL, deg_drain, None)
        plsc.subcore_barrier()

        pltpu.sync_copy(deg_sh.at[pl.ds(node0, NPT)],
                        dinv_v.at[pl.ds(node0, NPT)])
        for k in range(NPT // 16):
            d = dinv_v[pl.ds(node0 + k * 16, 16)] + 1.0
            dinv_v[pl.ds(node0 + k * 16, 16)] = _qrsqrt(d)
        pltpu.sync_copy(dinv_v.at[pl.ds(node0, NPT)],
                        dinv_sh.at[pl.ds(node0, NPT)])

        @pl.when(c == 0)
        def _():
            pltpu.sync_copy(dinv_v.at[pl.ds(node0, NPT)],
                            dinv_hbm.at[pl.ds(node0, NPT)])
        plsc.subcore_barrier()
        pltpu.sync_copy(dinv_sh, dinv_v)

        er0 = wid * ROWS_HALF
        pltpu.sync_copy(src_hbm.at[pl.ds(er0, ROWS_HALF)], srcv)
        pltpu.sync_copy(dst_hbm.at[pl.ds(er0, ROWS_HALF)], dstv)

        def b_step(r, carry):
            for j in range(8):
                s16 = srcv[r, pl.ds(j * 16, 16)]
                d16 = dstv[r, pl.ds(j * 16, 16)]
                xs = plsc.load_gather(x_v, [s16])
                dv = plsc.load_gather(dinv_v, [s16])
                idxv[r, pl.ds(j * 16, 16)] = xs * NP + d16
                valv[r, pl.ds(j * 16, 16)] = dv
            pltpu.async_copy(valv.at[r], b_sh.at[idxv.at[r]], sem, add=True)
            return carry
        lax.fori_loop(0, ROWS_HALF, b_step, None)

        @pl.when(c == 0)
        def _():
            def self_step(r, carry):
                for j in range(8):
                    nb = node0 + r * 128 + j * 16
                    n16 = lax.iota(I32, 16) + nb
                    xs = x_v[pl.ds(nb, 16)]
                    selfidx[r, pl.ds(j * 16, 16)] = xs * NP + n16
                    selfval[r, pl.ds(j * 16, 16)] = dinv_v[pl.ds(nb, 16)]
                pltpu.async_copy(selfval.at[r], b_sh.at[selfidx.at[r]], sem,
                                 add=True)
                return carry
            lax.fori_loop(0, 5, self_step, None)

        def b_drain(r, carry):
            pltpu.make_async_copy(valv.at[r], b_sh.at[idxv.at[r]], sem).wait()
            return carry
        lax.fori_loop(0, ROWS_HALF, b_drain, None)

        @pl.when(c == 0)
        def _():
            def self_drain(r, carry):
                pltpu.make_async_copy(selfval.at[r], b_sh.at[selfidx.at[r]],
                                      sem).wait()
                return carry
            lax.fori_loop(0, 5, self_drain, None)
        plsc.subcore_barrier()

        for cat in range(64):
            pltpu.async_copy(b_sh.at[pl.ds(cat * NP + node0, NPT)],
                             outB_hbm.at[c, cat, pl.ds(node0, NPT)], sem)
        for cat in range(64):
            pltpu.make_async_copy(b_sh.at[pl.ds(cat * NP + node0, NPT)],
                                  outB_hbm.at[c, cat, pl.ds(node0, NPT)],
                                  sem).wait()

    f = pl.kernel(
        body,
        out_type=[jax.ShapeDtypeStruct((NC, 64, NP), F32),
                  jax.ShapeDtypeStruct((NP,), F32)],
        mesh=mesh,
        compiler_params=pltpu.CompilerParams(needs_layout_passes=False),
        scratch_types=[
            pltpu.VMEM_SHARED((NP * 64,), F32),
            pltpu.VMEM_SHARED((NP,), F32),
            pltpu.VMEM_SHARED((NP,), F32),
            pltpu.VMEM((NP,), I32),
            pltpu.VMEM((NP,), F32),
            pltpu.VMEM((ROWS_ALL, 128), I32),
            pltpu.VMEM((ROWS_HALF, 128), I32),
            pltpu.VMEM((ROWS_HALF, 128), I32),
            pltpu.VMEM((ROWS_HALF, 128), I32),
            pltpu.VMEM((ROWS_HALF, 128), F32),
            pltpu.VMEM((5, 128), I32),
            pltpu.VMEM((5, 128), F32),
            pltpu.VMEM((1, 128), F32),
            pltpu.VMEM((2560,), F32),
            pltpu.SemaphoreType.DMA,
        ],
    )
    return f(src2d, dst2d, x)


FW = 64


def _sc_b(src2d, dst2d, g):
    mesh = plsc.VectorSubcoreMesh(core_axis_name="c", subcore_axis_name="s")

    def body(src_hbm, dst_hbm, g_hbm, outS_hbm,
             s_sh, srcv, dstv, rows0, rows1, rows2, rows3, zrows,
             sem0, sem1, sem2, sem3):
        c = lax.axis_index("c")
        s = lax.axis_index("s")
        wid = s * NC + c
        node0 = s * NPT
        bufs = (rows0, rows1, rows2, rows3)
        sems = (sem0, sem1, sem2, sem3)

        def zrow_step(r, carry):
            for j in range(FW // 16):
                zrows[r, pl.ds(j * 16, 16)] = jnp.zeros((16,), F32)
            return carry
        lax.fori_loop(0, 8, zrow_step, None)

        def zinit_step(k, carry):
            pltpu.sync_copy(zrows, s_sh.at[pl.ds(node0 + k * 8, 8)])
            return carry
        lax.fori_loop(0, NPT // 8, zinit_step, None)
        plsc.subcore_barrier()

        er0 = wid * ROWS_HALF
        pltpu.sync_copy(src_hbm.at[pl.ds(er0, ROWS_HALF)], srcv)
        pltpu.sync_copy(dst_hbm.at[pl.ds(er0, ROWS_HALF)], dstv)

        for jj in range(3):
            pltpu.async_copy(g_hbm.at[srcv.at[jj]], bufs[jj], sems[jj])

        def step(i, carry):
            r0 = i * 4
            for jj in range(4):
                r = r0 + jj
                pf = r + 3
                @pl.when(pf < ROWS_HALF)
                def _():
                    pltpu.async_copy(g_hbm.at[srcv.at[pf]], bufs[(jj + 3) % 4],
                                     sems[(jj + 3) % 4])
                pltpu.make_async_copy(g_hbm.at[srcv.at[r]], bufs[jj],
                                      sems[jj]).wait()
                pltpu.sync_copy(bufs[jj], s_sh.at[dstv.at[r]], add=True)
            return carry
        lax.fori_loop(0, ROWS_HALF // 4, step, None)
        plsc.subcore_barrier()

        for k in range(5):
            sl = pl.ds(node0 + k * 128, 128)
            pltpu.sync_copy(s_sh.at[sl], rows0)
            pltpu.sync_copy(rows0, outS_hbm.at[c, sl])

    f = pl.kernel(
        body,
        out_type=jax.ShapeDtypeStruct((NC, NP, FW), F32),
        mesh=mesh,
        compiler_params=pltpu.CompilerParams(needs_layout_passes=False,
                                             use_tc_tiling_on_sc=False),
        scratch_types=[
            pltpu.VMEM_SHARED((NP, FW), F32),
            pltpu.VMEM((ROWS_HALF, 128), I32),
            pltpu.VMEM((ROWS_HALF, 128), I32),
            pltpu.VMEM((128, FW), F32),
            pltpu.VMEM((128, FW), F32),
            pltpu.VMEM((128, FW), F32),
            pltpu.VMEM((128, FW), F32),
            pltpu.VMEM((8, FW), F32),
            pltpu.SemaphoreType.DMA,
            pltpu.SemaphoreType.DMA,
            pltpu.SemaphoreType.DMA,
            pltpu.SemaphoreType.DMA,
        ],
    )
    return f(src2d, dst2d, g)


def _tc1_body(b0_ref, b1_ref, dinv_ref, emb_ref, w1_ref, bias1_ref, w2_ref,
              g_ref, h1p_ref):
    i = pl.program_id(0)

    @pl.when(i == 0)
    def _():
        h1p_ref[...] = jnp.dot(emb_ref[...], w1_ref[...],
                               preferred_element_type=F32)
    Bt = b0_ref[0] + b1_ref[0]
    t = lax.dot_general(Bt, h1p_ref[...], (((0,), (0,)), ((), ())),
                        preferred_element_type=F32)
    t = jax.nn.relu(dinv_ref[...] * t + bias1_ref[...])
    g_ref[...] = jnp.dot(t, w2_ref[...],
                         preferred_element_type=F32) * dinv_ref[...]


def _tc1(outB, dinv2d, emb_pad, W1, b1, W2):
    BLK = 2048
    return pl.pallas_call(
        _tc1_body,
        grid=(NP // BLK,),
        in_specs=[
            pl.BlockSpec((1, 64, BLK), lambda i: (0, 0, i)),
            pl.BlockSpec((1, 64, BLK), lambda i: (1, 0, i)),
            pl.BlockSpec((BLK, 1), lambda i: (i, 0)),
            pl.BlockSpec((64, EMB), lambda i: (0, 0)),
            pl.BlockSpec((EMB, HID), lambda i: (0, 0)),
            pl.BlockSpec((1, HID), lambda i: (0, 0)),
            pl.BlockSpec((HID, OUT), lambda i: (0, 0)),
        ],
        out_specs=pl.BlockSpec((BLK, OUT), lambda i: (i, 0)),
        out_shape=jax.ShapeDtypeStruct((NP, OUT), F32),
        scratch_shapes=[pltpu.VMEM((64, HID), F32)],
    )(outB, outB, dinv2d, emb_pad, W1, b1, W2)


def _tc2_body(s0_ref, s1_ref, g_ref, dinv_ref, b2_ref, wout_ref, bout_ref,
              batch_ref, out_ref, macc, cacc):
    i = pl.program_id(0)
    ng = pl.num_programs(0)

    @pl.when(i == 0)
    def _():
        macc[...] = jnp.zeros_like(macc)
        cacc[...] = jnp.zeros_like(cacc)
    sm = s0_ref[0] + s1_ref[0] + g_ref[...]
    a2 = dinv_ref[...] * sm + b2_ref[...]
    h3 = jax.nn.relu(jnp.dot(a2, wout_ref[...],
                             preferred_element_type=F32) + bout_ref[...])
    M = (batch_ref[...] == lax.broadcasted_iota(I32, (1, NUM_GRAPHS), 1)
         ).astype(F32)
    macc[...] += lax.dot_general(M, h3, (((0,), (0,)), ((), ())),
                                 preferred_element_type=F32)
    cacc[...] += lax.dot_general(M, jnp.ones_like(h3), (((0,), (0,)), ((), ())),
                                 preferred_element_type=F32)

    @pl.when(i == ng - 1)
    def _():
        out_ref[...] = macc[...] / jnp.maximum(cacc[...], 1.0)


def _tc2(outS, G, dinv2d, b2, Wout, bout, batch2d):
    BLK = 2048
    return pl.pallas_call(
        _tc2_body,
        grid=(NP // BLK,),
        in_specs=[
            pl.BlockSpec((1, BLK, OUT), lambda i: (0, i, 0)),
            pl.BlockSpec((1, BLK, OUT), lambda i: (1, i, 0)),
            pl.BlockSpec((BLK, OUT), lambda i: (i, 0)),
            pl.BlockSpec((BLK, 1), lambda i: (i, 0)),
            pl.BlockSpec((1, OUT), lambda i: (0, 0)),
            pl.BlockSpec((OUT, OUT), lambda i: (0, 0)),
            pl.BlockSpec((1, OUT), lambda i: (0, 0)),
            pl.BlockSpec((BLK, 1), lambda i: (i, 0)),
        ],
        out_specs=pl.BlockSpec((NUM_GRAPHS, NUM_GRAPHS), lambda i: (0, 0)),
        out_shape=jax.ShapeDtypeStruct((NUM_GRAPHS, NUM_GRAPHS), F32),
        scratch_shapes=[pltpu.VMEM((NUM_GRAPHS, NUM_GRAPHS), F32),
                        pltpu.VMEM((NUM_GRAPHS, NUM_GRAPHS), F32)],
    )(outS, outS, G, dinv2d, b2, Wout, bout, batch2d)


def kernel(x, edge_index, edge_attr, batch, emb_table, W1, b1, W2, b2,
           Wout, bout):
    del edge_attr
    x = x.astype(I32)
    src = edge_index[0].astype(I32)
    dst = edge_index[1].astype(I32)
    pad = EP - E
    pidx = jnp.arange(pad, dtype=I32)
    src2d = jnp.concatenate([src, pidx % N]).reshape(EROWS, 128)
    dst2d = jnp.concatenate([dst, N + pidx % (NP - N)]).reshape(EROWS, 128)

    outB, dinv = _sc_a(src2d, dst2d, x)
    dinv2d = dinv.reshape(NP, 1)

    emb_pad = jnp.zeros((64, EMB), F32).at[:NUM_CAT].set(emb_table)
    g = _tc1(outB, dinv2d, emb_pad, W1, b1.reshape(1, HID), W2)

    outS = _sc_b(src2d, dst2d, g)

    batch_pad = jnp.concatenate([batch.astype(I32),
                                 jnp.full((NP - N,), NUM_GRAPHS, I32)])
    out = _tc2(outS, g, dinv2d, b2.reshape(1, OUT), Wout,
               bout.reshape(1, OUT), batch_pad.reshape(NP, 1))
    return out


# speedup vs baseline: 1.3643x; 1.0189x over previous
import jax
import jax.numpy as jnp
from jax import lax
from jax.experimental import pallas as pl
from jax.experimental.pallas import tpu as pltpu
from jax.experimental.pallas import tpu_sc as plsc

N = 10000
NP = 10240
E = 320000
EROWS = E // 128
NUM_CAT = 43
EMB = 128
HID = 256
OUT = 64
NUM_GRAPHS = 64
F32 = jnp.float32
I32 = jnp.int32

NC, NS = 2, 16
NW = NC * NS
ROWS_ALL = EROWS // NS
ROWS_HALF = EROWS // NW
EXTRA0 = NS * ROWS_ALL
NPT = NP // NS
JUNK_NODE = NP - 1


def _qrsqrt(d):
    i = plsc.bitcast(d, I32)
    i = jnp.int32(0x5F3759DF) - (i >> 1)
    y = plsc.bitcast(i, F32)
    for _ in range(3):
        y = y * (1.5 - 0.5 * d * y * y)
    return y


def _sc_a(ei3, x):
    mesh = plsc.VectorSubcoreMesh(core_axis_name="c", subcore_axis_name="s")

    def body(ei_hbm, x_hbm, outB_hbm, dinv_hbm,
             b_sh, deg_sh, dinv_sh,
             x_v, dinv_v, dstv_all, srcv, dstv, idxv, valv,
             selfidx, selfval, ones_row, zeros_v, sem):
        c = lax.axis_index("c")
        s = lax.axis_index("s")
        wid = s * NC + c
        node0 = s * NPT

        for j in range(8):
            ones_row[0, pl.ds(j * 16, 16)] = jnp.ones((16,), F32)

        def zero_step(r, carry):
            zeros_v[pl.ds(r * 16, 16)] = jnp.zeros((16,), F32)
            return carry
        lax.fori_loop(0, 160, zero_step, None)

        pltpu.sync_copy(x_hbm, x_v.at[pl.ds(0, N)])
        for j in range(15):
            x_v[pl.ds(N + j * 16, 16)] = jnp.zeros((16,), I32)

        pltpu.sync_copy(zeros_v.at[pl.ds(0, NPT)],
                        deg_sh.at[pl.ds(node0, NPT)])
        for k in range(16):
            pltpu.sync_copy(zeros_v, b_sh.at[pl.ds(node0 * 64 + k * 2560,
                                                   2560)])
        plsc.subcore_barrier()

        pltpu.sync_copy(ei_hbm.at[1, pl.ds(s * ROWS_ALL, ROWS_ALL)],
                        dstv_all.at[pl.ds(0, ROWS_ALL)])

        @pl.when(s < EROWS - NS * ROWS_ALL)
        def _():
            pltpu.sync_copy(ei_hbm.at[1, pl.ds(EXTRA0 + s, 1)],
                            dstv_all.at[pl.ds(ROWS_ALL, 1)])
        nr_all = jnp.where(s < EROWS - NS * ROWS_ALL, ROWS_ALL + 1, ROWS_ALL)

        def deg_step(r, carry):
            pltpu.async_copy(ones_row.at[0], deg_sh.at[dstv_all.at[r]], sem,
                             add=True)
            return carry
        lax.fori_loop(0, nr_all, deg_step, None)

        def deg_drain(r, carry):
            pltpu.make_async_copy(ones_row.at[0], deg_sh.at[dstv_all.at[r]],
                                  sem).wait()
            return carry
        lax.fori_loop(0, nr_all, deg_drain, None)
        plsc.subcore_barrier()

        pltpu.sync_copy(deg_sh.at[pl.ds(node0, NPT)],
                        dinv_v.at[pl.ds(node0, NPT)])
        for k in range(NPT // 16):
            d = dinv_v[pl.ds(node0 + k * 16, 16)] + 1.0
            dinv_v[pl.ds(node0 + k * 16, 16)] = _qrsqrt(d)
        pltpu.sync_copy(dinv_v.at[pl.ds(node0, NPT)],
                        dinv_sh.at[pl.ds(node0, NPT)])

        @pl.when(c == 0)
        def _():
            pltpu.sync_copy(dinv_v.at[pl.ds(node0, NPT)],
                            dinv_hbm.at[pl.ds(node0, NPT)])
        plsc.subcore_barrier()
        pltpu.sync_copy(dinv_sh, dinv_v)

        er0 = wid * ROWS_HALF
        pltpu.sync_copy(ei_hbm.at[0, pl.ds(er0, ROWS_HALF)],
                        srcv.at[pl.ds(0, ROWS_HALF)])
        pltpu.sync_copy(ei_hbm.at[1, pl.ds(er0, ROWS_HALF)],
                        dstv.at[pl.ds(0, ROWS_HALF)])

        @pl.when(wid < EROWS - NW * ROWS_HALF)
        def _():
            pltpu.sync_copy(ei_hbm.at[0, pl.ds(EXTRA0 + wid, 1)],
                            srcv.at[pl.ds(ROWS_HALF, 1)])
            pltpu.sync_copy(ei_hbm.at[1, pl.ds(EXTRA0 + wid, 1)],
                            dstv.at[pl.ds(ROWS_HALF, 1)])
        nr = jnp.where(wid < EROWS - NW * ROWS_HALF, ROWS_HALF + 1, ROWS_HALF)

        def b_step(r, carry):
            for j in range(8):
                s16 = srcv[r, pl.ds(j * 16, 16)]
                d16 = dstv[r, pl.ds(j * 16, 16)]
                xs = plsc.load_gather(x_v, [s16])
                dv = plsc.load_gather(dinv_v, [s16])
                idxv[r, pl.ds(j * 16, 16)] = xs * NP + d16
                valv[r, pl.ds(j * 16, 16)] = dv
            pltpu.async_copy(valv.at[r], b_sh.at[idxv.at[r]], sem, add=True)
            return carry
        lax.fori_loop(0, nr, b_step, None)

        @pl.when(c == 0)
        def _():
            def self_step(r, carry):
                for j in range(8):
                    nb = node0 + r * 128 + j * 16
                    n16 = lax.iota(I32, 16) + nb
                    xs = x_v[pl.ds(nb, 16)]
                    selfidx[r, pl.ds(j * 16, 16)] = xs * NP + n16
                    selfval[r, pl.ds(j * 16, 16)] = dinv_v[pl.ds(nb, 16)]
                pltpu.async_copy(selfval.at[r], b_sh.at[selfidx.at[r]], sem,
                                 add=True)
                return carry
            lax.fori_loop(0, 5, self_step, None)

        def b_drain(r, carry):
            pltpu.make_async_copy(valv.at[r], b_sh.at[idxv.at[r]], sem).wait()
            return carry
        lax.fori_loop(0, nr, b_drain, None)

        @pl.when(c == 0)
        def _():
            def self_drain(r, carry):
                pltpu.make_async_copy(selfval.at[r], b_sh.at[selfidx.at[r]],
                                      sem).wait()
                return carry
            lax.fori_loop(0, 5, self_drain, None)
        plsc.subcore_barrier()

        for cat in range(64):
            pltpu.async_copy(b_sh.at[pl.ds(cat * NP + node0, NPT)],
                             outB_hbm.at[c, cat, pl.ds(node0, NPT)], sem)
        for cat in range(64):
            pltpu.make_async_copy(b_sh.at[pl.ds(cat * NP + node0, NPT)],
                                  outB_hbm.at[c, cat, pl.ds(node0, NPT)],
                                  sem).wait()

    f = pl.kernel(
        body,
        out_type=[jax.ShapeDtypeStruct((NC, 64, NP), F32),
                  jax.ShapeDtypeStruct((NP,), F32)],
        mesh=mesh,
        compiler_params=pltpu.CompilerParams(needs_layout_passes=False,
                                             use_tc_tiling_on_sc=False),
        scratch_types=[
            pltpu.VMEM_SHARED((NP * 64,), F32),
            pltpu.VMEM_SHARED((NP,), F32),
            pltpu.VMEM_SHARED((NP,), F32),
            pltpu.VMEM((NP,), I32),
            pltpu.VMEM((NP,), F32),
            pltpu.VMEM((ROWS_ALL + 1, 128), I32),
            pltpu.VMEM((ROWS_HALF + 1, 128), I32),
            pltpu.VMEM((ROWS_HALF + 1, 128), I32),
            pltpu.VMEM((ROWS_HALF + 1, 128), I32),
            pltpu.VMEM((ROWS_HALF + 1, 128), F32),
            pltpu.VMEM((5, 128), I32),
            pltpu.VMEM((5, 128), F32),
            pltpu.VMEM((1, 128), F32),
            pltpu.VMEM((2560,), F32),
            pltpu.SemaphoreType.DMA,
        ],
    )
    return f(ei3, x)


FW = 64


def _sc_b(ei3, g):
    mesh = plsc.VectorSubcoreMesh(core_axis_name="c", subcore_axis_name="s")

    def body(ei_hbm, g_hbm, outS_hbm,
             s_sh, srcv, dstv, rows0, rows1, rows2, rows3, zrows,
             sem0, sem1, sem2, sem3):
        c = lax.axis_index("c")
        s = lax.axis_index("s")
        wid = s * NC + c
        node0 = s * NPT
        bufs = (rows0, rows1, rows2, rows3)
        sems = (sem0, sem1, sem2, sem3)

        def zrow_step(r, carry):
            for j in range(FW // 16):
                zrows[r, pl.ds(j * 16, 16)] = jnp.zeros((16,), F32)
            return carry
        lax.fori_loop(0, 8, zrow_step, None)

        def zinit_step(k, carry):
            pltpu.sync_copy(zrows, s_sh.at[pl.ds(node0 + k * 8, 8)])
            return carry
        lax.fori_loop(0, NPT // 8, zinit_step, None)
        plsc.subcore_barrier()

        er0 = wid * ROWS_HALF
        pltpu.sync_copy(ei_hbm.at[0, pl.ds(er0, ROWS_HALF)],
                        srcv.at[pl.ds(0, ROWS_HALF)])
        pltpu.sync_copy(ei_hbm.at[1, pl.ds(er0, ROWS_HALF)],
                        dstv.at[pl.ds(0, ROWS_HALF)])

        @pl.when(wid < EROWS - NW * ROWS_HALF)
        def _():
            pltpu.sync_copy(ei_hbm.at[0, pl.ds(EXTRA0 + wid, 1)],
                            srcv.at[pl.ds(ROWS_HALF, 1)])
            pltpu.sync_copy(ei_hbm.at[1, pl.ds(EXTRA0 + wid, 1)],
                            dstv.at[pl.ds(ROWS_HALF, 1)])
        nr = jnp.where(wid < EROWS - NW * ROWS_HALF, ROWS_HALF + 1, ROWS_HALF)

        MAIN = (ROWS_HALF - 2) // 4 * 4
        for jj in range(3):
            pltpu.async_copy(g_hbm.at[srcv.at[jj]], bufs[jj], sems[jj])

        def step(i, carry):
            r0 = i * 4
            for jj in range(4):
                r = r0 + jj
                pf = r + 3
                @pl.when(pf < nr)
                def _():
                    pltpu.async_copy(g_hbm.at[srcv.at[pf]], bufs[(jj + 3) % 4],
                                     sems[(jj + 3) % 4])
                pltpu.make_async_copy(g_hbm.at[srcv.at[r]], bufs[jj],
                                      sems[jj]).wait()
                pltpu.sync_copy(bufs[jj], s_sh.at[dstv.at[r]], add=True)
            return carry
        lax.fori_loop(0, MAIN // 4, step, None)
        for r in range(MAIN, ROWS_HALF):
            pltpu.make_async_copy(g_hbm.at[srcv.at[r]], bufs[r % 4],
                                  sems[r % 4]).wait()
            pltpu.sync_copy(bufs[r % 4], s_sh.at[dstv.at[r]], add=True)

        @pl.when(nr > ROWS_HALF)
        def _():
            r = ROWS_HALF
            pltpu.make_async_copy(g_hbm.at[srcv.at[r]], bufs[r % 4],
                                  sems[r % 4]).wait()
            pltpu.sync_copy(bufs[r % 4], s_sh.at[dstv.at[r]], add=True)
        plsc.subcore_barrier()

        for k in range(5):
            sl = pl.ds(node0 + k * 128, 128)
            pltpu.sync_copy(s_sh.at[sl], rows0)
            pltpu.sync_copy(rows0, outS_hbm.at[c, sl])

    f = pl.kernel(
        body,
        out_type=jax.ShapeDtypeStruct((NC, NP, FW), F32),
        mesh=mesh,
        compiler_params=pltpu.CompilerParams(needs_layout_passes=False,
                                             use_tc_tiling_on_sc=False),
        scratch_types=[
            pltpu.VMEM_SHARED((NP, FW), F32),
            pltpu.VMEM((ROWS_HALF + 1, 128), I32),
            pltpu.VMEM((ROWS_HALF + 1, 128), I32),
            pltpu.VMEM((128, FW), F32),
            pltpu.VMEM((128, FW), F32),
            pltpu.VMEM((128, FW), F32),
            pltpu.VMEM((128, FW), F32),
            pltpu.VMEM((8, FW), F32),
            pltpu.SemaphoreType.DMA,
            pltpu.SemaphoreType.DMA,
            pltpu.SemaphoreType.DMA,
            pltpu.SemaphoreType.DMA,
        ],
    )
    return f(ei3, g)


def _tc1_body(b0_ref, b1_ref, dinv_ref, emb_ref, w1_ref, bias1_ref, w2_ref,
              g_ref, h1p_ref):
    i = pl.program_id(0)

    @pl.when(i == 0)
    def _():
        h1p_ref[...] = jnp.dot(emb_ref[...], w1_ref[...],
                               preferred_element_type=F32)
    Bt = b0_ref[0] + b1_ref[0]
    t = lax.dot_general(Bt, h1p_ref[...], (((0,), (0,)), ((), ())),
                        preferred_element_type=F32)
    t = jax.nn.relu(dinv_ref[...] * t + bias1_ref[...])
    g_ref[...] = jnp.dot(t, w2_ref[...],
                         preferred_element_type=F32) * dinv_ref[...]


def _tc1(outB, dinv2d, emb_pad, W1, b1, W2):
    BLK = 2048
    return pl.pallas_call(
        _tc1_body,
        grid=(NP // BLK,),
        in_specs=[
            pl.BlockSpec((1, 64, BLK), lambda i: (0, 0, i)),
            pl.BlockSpec((1, 64, BLK), lambda i: (1, 0, i)),
            pl.BlockSpec((BLK, 1), lambda i: (i, 0)),
            pl.BlockSpec((64, EMB), lambda i: (0, 0)),
            pl.BlockSpec((EMB, HID), lambda i: (0, 0)),
            pl.BlockSpec((1, HID), lambda i: (0, 0)),
            pl.BlockSpec((HID, OUT), lambda i: (0, 0)),
        ],
        out_specs=pl.BlockSpec((BLK, OUT), lambda i: (i, 0)),
        out_shape=jax.ShapeDtypeStruct((NP, OUT), F32),
        scratch_shapes=[pltpu.VMEM((64, HID), F32)],
    )(outB, outB, dinv2d, emb_pad, W1, b1, W2)


def _tc2_body(s0_ref, s1_ref, g_ref, dinv_ref, b2_ref, wout_ref, bout_ref,
              batch_ref, out_ref, macc, cacc):
    i = pl.program_id(0)
    ng = pl.num_programs(0)

    @pl.when(i == 0)
    def _():
        macc[...] = jnp.zeros_like(macc)
        cacc[...] = jnp.zeros_like(cacc)
    sm = s0_ref[0] + s1_ref[0] + g_ref[...]
    a2 = dinv_ref[...] * sm + b2_ref[...]
    h3 = jax.nn.relu(jnp.dot(a2, wout_ref[...],
                             preferred_element_type=F32) + bout_ref[...])
    M = (batch_ref[...] == lax.broadcasted_iota(I32, (1, NUM_GRAPHS), 1)
         ).astype(F32)
    macc[...] += lax.dot_general(M, h3, (((0,), (0,)), ((), ())),
                                 preferred_element_type=F32)
    cacc[...] += lax.dot_general(M, jnp.ones_like(h3), (((0,), (0,)), ((), ())),
                                 preferred_element_type=F32)

    @pl.when(i == ng - 1)
    def _():
        out_ref[...] = macc[...] / jnp.maximum(cacc[...], 1.0)


def _tc2(outS, G, dinv2d, b2, Wout, bout, batch2d):
    BLK = 2048
    return pl.pallas_call(
        _tc2_body,
        grid=(NP // BLK,),
        in_specs=[
            pl.BlockSpec((1, BLK, OUT), lambda i: (0, i, 0)),
            pl.BlockSpec((1, BLK, OUT), lambda i: (1, i, 0)),
            pl.BlockSpec((BLK, OUT), lambda i: (i, 0)),
            pl.BlockSpec((BLK, 1), lambda i: (i, 0)),
            pl.BlockSpec((1, OUT), lambda i: (0, 0)),
            pl.BlockSpec((OUT, OUT), lambda i: (0, 0)),
            pl.BlockSpec((1, OUT), lambda i: (0, 0)),
            pl.BlockSpec((BLK, 1), lambda i: (i, 0)),
        ],
        out_specs=pl.BlockSpec((NUM_GRAPHS, NUM_GRAPHS), lambda i: (0, 0)),
        out_shape=jax.ShapeDtypeStruct((NUM_GRAPHS, NUM_GRAPHS), F32),
        scratch_shapes=[pltpu.VMEM((NUM_GRAPHS, NUM_GRAPHS), F32),
                        pltpu.VMEM((NUM_GRAPHS, NUM_GRAPHS), F32)],
    )(outS, outS, G, dinv2d, b2, Wout, bout, batch2d)


def kernel(x, edge_index, edge_attr, batch, emb_table, W1, b1, W2, b2,
           Wout, bout):
    del edge_attr
    x = x.astype(I32)
    ei3 = edge_index.astype(I32).reshape(2, EROWS, 128)

    outB, dinv = _sc_a(ei3, x)
    dinv2d = dinv.reshape(NP, 1)

    emb_pad = jnp.zeros((64, EMB), F32).at[:NUM_CAT].set(emb_table)
    g = _tc1(outB, dinv2d, emb_pad, W1, b1.reshape(1, HID), W2)

    outS = _sc_b(ei3, g)

    batch_pad = jnp.concatenate([batch.astype(I32),
                                 jnp.full((NP - N,), NUM_GRAPHS, I32)])
    out = _tc2(outS, g, dinv2d, b2.reshape(1, OUT), Wout,
               bout.reshape(1, OUT), batch_pad.reshape(NP, 1))
    return out
